# baseline (speedup 1.0000x reference)
import functools

import jax
import jax.numpy as jnp
from jax import lax
from jax.experimental import pallas as pl
from jax.experimental.pallas import tpu as pltpu
from jax.experimental.pallas import tpu_sc as plsc

BATCH = 16384
DIM = 32
PACK = 128 // DIM
NC = 2
NS = 16
NW = NC * NS
BPW = BATCH // NW
NCHUNK = 2
CHUNK = BPW // NCHUNK


def _sc_gather_kernel(ut_hbm, it_hbm, ui_hbm, ii_hbm, uo_hbm, io_hbm,
                      uidx_v, urows_v, iidx_v, irows_v, usem, isem):
    wid = lax.axis_index("s") * NC + lax.axis_index("c")
    for c in range(NCHUNK):
        base = wid * BPW + c * CHUNK
        pltpu.sync_copy(ui_hbm.at[pl.ds(base, CHUNK)], uidx_v)
        pltpu.sync_copy(ii_hbm.at[pl.ds(base, CHUNK)], iidx_v)
        ucp = pltpu.async_copy(ut_hbm.at[uidx_v], urows_v, usem)
        icp = pltpu.async_copy(it_hbm.at[iidx_v], irows_v, isem)
        ucp.wait()
        icp.wait()
        pltpu.sync_copy(urows_v, uo_hbm.at[pl.ds(base, CHUNK)])
        pltpu.sync_copy(irows_v, io_hbm.at[pl.ds(base, CHUNK)])


def _sc_gather(user_packed, item_packed, user_pidx, item_pidx):
    mesh = plsc.VectorSubcoreMesh(core_axis_name="c", subcore_axis_name="s")
    run = functools.partial(
        pl.kernel,
        mesh=mesh,
        out_type=(
            jax.ShapeDtypeStruct((BATCH, 128), jnp.float32),
            jax.ShapeDtypeStruct((BATCH, 128), jnp.float32),
        ),
        scratch_types=[
            pltpu.VMEM((CHUNK,), jnp.int32),
            pltpu.VMEM((CHUNK, 128), jnp.float32),
            pltpu.VMEM((CHUNK,), jnp.int32),
            pltpu.VMEM((CHUNK, 128), jnp.float32),
            pltpu.SemaphoreType.DMA,
            pltpu.SemaphoreType.DMA,
        ],
    )(_sc_gather_kernel)
    return run(user_packed, item_packed, user_pidx, item_pidx)


TC_GRID = 8
TC_BLK = BATCH // TC_GRID


def _tc_fuse_kernel(up_ref, ip_ref, uoff_ref, ioff_ref, ue_ref, ie_ref,
                    au_ref, ai_ref, lu_ref, li_ref):
    i = pl.program_id(0)
    up = up_ref[...]
    ip = ip_ref[...]
    uoff = uoff_ref[...]
    ioff = ioff_ref[...]
    ue = ue_ref[...]
    ie = ie_ref[...]
    ua = jnp.zeros((TC_BLK, DIM), jnp.float32)
    ia = jnp.zeros((TC_BLK, DIM), jnp.float32)
    for k in range(PACK):
        ua = ua + jnp.where(uoff == k, up[:, k * DIM:(k + 1) * DIM], 0.0)
        ia = ia + jnp.where(ioff == k, ip[:, k * DIM:(k + 1) * DIM], 0.0)
    au_ref[...] = ue + ua
    ai_ref[...] = ie + ia

    @pl.when(i == 0)
    def _():
        lu_ref[...] = jnp.zeros((1, 1), jnp.float32)
        li_ref[...] = jnp.zeros((1, 1), jnp.float32)

    inv_n = 1.0 / float(BATCH * DIM)
    lu_ref[...] += (jnp.sum((ua - ie) ** 2) * inv_n).reshape(1, 1)
    li_ref[...] += (jnp.sum((ia - ue) ** 2) * inv_n).reshape(1, 1)


def _tc_fuse(user_packed_rows, item_packed_rows, user_off, item_off,
             user_embedding, item_embedding):
    return pl.pallas_call(
        _tc_fuse_kernel,
        grid=(TC_GRID,),
        in_specs=[
            pl.BlockSpec((TC_BLK, 128), lambda i: (i, 0)),
            pl.BlockSpec((TC_BLK, 128), lambda i: (i, 0)),
            pl.BlockSpec((TC_BLK, 1), lambda i: (i, 0)),
            pl.BlockSpec((TC_BLK, 1), lambda i: (i, 0)),
            pl.BlockSpec((TC_BLK, DIM), lambda i: (i, 0)),
            pl.BlockSpec((TC_BLK, DIM), lambda i: (i, 0)),
        ],
        out_specs=(
            pl.BlockSpec((TC_BLK, DIM), lambda i: (i, 0)),
            pl.BlockSpec((TC_BLK, DIM), lambda i: (i, 0)),
            pl.BlockSpec((1, 1), lambda i: (0, 0)),
            pl.BlockSpec((1, 1), lambda i: (0, 0)),
        ),
        out_shape=(
            jax.ShapeDtypeStruct((BATCH, DIM), jnp.float32),
            jax.ShapeDtypeStruct((BATCH, DIM), jnp.float32),
            jax.ShapeDtypeStruct((1, 1), jnp.float32),
            jax.ShapeDtypeStruct((1, 1), jnp.float32),
        ),
    )(user_packed_rows, item_packed_rows, user_off, item_off,
      user_embedding, item_embedding)


def kernel(user_indices, item_indices, user_embedding, item_embedding,
           user_table, item_table):
    user_packed = user_table.reshape(-1, 128)
    item_packed = item_table.reshape(-1, 128)
    user_pidx = lax.shift_right_logical(user_indices, 2)
    item_pidx = lax.shift_right_logical(item_indices, 2)
    user_off = jnp.bitwise_and(user_indices, PACK - 1).reshape(BATCH, 1)
    item_off = jnp.bitwise_and(item_indices, PACK - 1).reshape(BATCH, 1)
    user_rows, item_rows = _sc_gather(user_packed, item_packed,
                                      user_pidx, item_pidx)
    augmented_user, augmented_item, lu, li = _tc_fuse(
        user_rows, item_rows, user_off, item_off,
        user_embedding, item_embedding)
    return (augmented_user, augmented_item, lu[0, 0], li[0, 0])


# speedup vs baseline: 2.8341x; 2.8341x over previous
import dataclasses
import functools

import jax
import jax.numpy as jnp
from jax import lax
from jax.experimental import pallas as pl
from jax.experimental.pallas import tpu as pltpu
from jax.experimental.pallas import tpu_sc as plsc

BATCH = 16384
DIM = 32
NC = 2
NS = 16
NW = NC * NS
BPW = BATCH // NW
LANES = 16
NOCT = DIM // 8


def _sc_fused_kernel(ut_hbm, it_hbm, ui_hbm, ii_hbm, ue_hbm, ie_hbm,
                     au_hbm, ai_hbm, lu_hbm, li_hbm,
                     uidx_v, iidx_v, tu0, tu1, ti0, ti1,
                     urows_v, irows_v, uemb_v, iemb_v, uacc_v, iacc_v,
                     su0, su1, si0, si1, esem):
    wid = lax.axis_index("s") * NC + lax.axis_index("c")
    base = wid * BPW
    pltpu.sync_copy(ui_hbm.at[pl.ds(base, BPW)], uidx_v)
    pltpu.sync_copy(ii_hbm.at[pl.ds(base, BPW)], iidx_v)

    ue_cp = pltpu.async_copy(ue_hbm.at[:, pl.ds(base, BPW)], uemb_v, esem)
    ie_cp = pltpu.async_copy(ie_hbm.at[:, pl.ds(base, BPW)], iemb_v, esem)

    def prefetch(ru, ri, tu, ti, su, si):
        bu = pl.multiple_of((ru >> 7) << 7, 128)
        bi = pl.multiple_of((ri >> 7) << 7, 128)
        for k in range(NOCT):
            pltpu.async_copy(ut_hbm.at[pl.ds(8 * k, 8), pl.ds(bu, 128)],
                             tu.at[pl.ds(8 * k, 8), :], su)
            pltpu.async_copy(it_hbm.at[pl.ds(8 * k, 8), pl.ds(bi, 128)],
                             ti.at[pl.ds(8 * k, 8), :], si)

    def drain(tbuf, sem):
        pltpu.make_async_copy(ut_hbm.at[:, pl.ds(0, 128)], tbuf, sem).wait()

    dvec0 = lax.broadcasted_iota(jnp.int32, (LANES,), 0)
    dvec1 = dvec0 + LANES

    def extract(j, ridx, tbuf, rows):
        rmod = jnp.bitwise_and(ridx, 127)
        lvec = jnp.full((LANES,), rmod, jnp.int32)
        jvec = jnp.full((LANES,), j, jnp.int32)
        v0 = plsc.load_gather(tbuf, [dvec0, lvec])
        v1 = plsc.load_gather(tbuf, [dvec1, lvec])
        plsc.store_scatter(rows, [dvec0, jvec], v0)
        plsc.store_scatter(rows, [dvec1, jvec], v1)

    tu = (tu0, tu1)
    ti = (ti0, ti1)
    su = (su0, su1)
    si = (si0, si1)

    @pl.loop(0, BPW // LANES)
    def _(cidx):
        vu = uidx_v[pl.ds(cidx * LANES, LANES)]
        vi = iidx_v[pl.ds(cidx * LANES, LANES)]
        prefetch(vu[0], vi[0], tu0, ti0, su0, si0)
        prefetch(vu[1], vi[1], tu1, ti1, su1, si1)
        for l in range(LANES):
            s = l & 1
            j = cidx * LANES + l
            drain(tu[s], su[s])
            drain(ti[s], si[s])
            extract(j, vu[l], tu[s], urows_v)
            extract(j, vi[l], ti[s], irows_v)
            if l + 2 < LANES:
                prefetch(vu[l + 2], vi[l + 2], tu[s], ti[s], su[s], si[s])

    ue_cp.wait()
    ie_cp.wait()

    uacc_v[...] = jnp.zeros((LANES,), jnp.float32)
    iacc_v[...] = jnp.zeros((LANES,), jnp.float32)

    @pl.loop(0, DIM)
    def _(d):
        @pl.loop(0, BPW // LANES)
        def _(c):
            cs = pl.ds(c * LANES, LANES)
            ug = urows_v.at[d, cs][...]
            ig = irows_v.at[d, cs][...]
            ue = uemb_v.at[d, cs][...]
            ie = iemb_v.at[d, cs][...]
            uemb_v.at[d, cs][...] = ue + ug
            iemb_v.at[d, cs][...] = ie + ig
            du = ug - ie
            di = ig - ue
            uacc_v[...] += du * du
            iacc_v[...] += di * di

    pltpu.sync_copy(uemb_v, au_hbm.at[:, pl.ds(base, BPW)])
    pltpu.sync_copy(iemb_v, ai_hbm.at[:, pl.ds(base, BPW)])
    pltpu.sync_copy(uacc_v, lu_hbm.at[wid])
    pltpu.sync_copy(iacc_v, li_hbm.at[wid])


def _sc_fused(user_table_t, item_table_t, user_indices, item_indices,
              user_emb_t, item_emb_t):
    mesh = plsc.VectorSubcoreMesh(core_axis_name="c", subcore_axis_name="s")
    cp = pltpu.CompilerParams()
    if "needs_layout_passes" in pltpu.CompilerParams.__dataclass_fields__:
        cp = dataclasses.replace(cp, needs_layout_passes=False)
    run = functools.partial(
        pl.kernel,
        mesh=mesh,
        compiler_params=cp,
        out_type=(
            jax.ShapeDtypeStruct((DIM, BATCH), jnp.float32),
            jax.ShapeDtypeStruct((DIM, BATCH), jnp.float32),
            jax.ShapeDtypeStruct((NW, LANES), jnp.float32),
            jax.ShapeDtypeStruct((NW, LANES), jnp.float32),
        ),
        scratch_types=[
            pltpu.VMEM((BPW,), jnp.int32),
            pltpu.VMEM((BPW,), jnp.int32),
            pltpu.VMEM((DIM, 128), jnp.float32),
            pltpu.VMEM((DIM, 128), jnp.float32),
            pltpu.VMEM((DIM, 128), jnp.float32),
            pltpu.VMEM((DIM, 128), jnp.float32),
            pltpu.VMEM((DIM, BPW), jnp.float32),
            pltpu.VMEM((DIM, BPW), jnp.float32),
            pltpu.VMEM((DIM, BPW), jnp.float32),
            pltpu.VMEM((DIM, BPW), jnp.float32),
            pltpu.VMEM((LANES,), jnp.float32),
            pltpu.VMEM((LANES,), jnp.float32),
            pltpu.SemaphoreType.DMA,
            pltpu.SemaphoreType.DMA,
            pltpu.SemaphoreType.DMA,
            pltpu.SemaphoreType.DMA,
            pltpu.SemaphoreType.DMA,
        ],
    )(_sc_fused_kernel)
    return run(user_table_t, item_table_t, user_indices, item_indices,
               user_emb_t, item_emb_t)


def _tc_reduce_kernel(lu_ref, li_ref, luo_ref, lio_ref):
    inv_n = 1.0 / float(BATCH * DIM)
    luo_ref[...] = (jnp.sum(lu_ref[...]) * inv_n).reshape(1, 1)
    lio_ref[...] = (jnp.sum(li_ref[...]) * inv_n).reshape(1, 1)


def _tc_reduce(lu_part, li_part):
    return pl.pallas_call(
        _tc_reduce_kernel,
        out_shape=(
            jax.ShapeDtypeStruct((1, 1), jnp.float32),
            jax.ShapeDtypeStruct((1, 1), jnp.float32),
        ),
    )(lu_part, li_part)


def kernel(user_indices, item_indices, user_embedding, item_embedding,
           user_table, item_table):
    au_t, ai_t, lu_part, li_part = _sc_fused(
        user_table.T, item_table.T, user_indices, item_indices,
        user_embedding.T, item_embedding.T)
    lu, li = _tc_reduce(lu_part, li_part)
    return (au_t.T, ai_t.T, lu[0, 0], li[0, 0])
